# baseline (speedup 1.0000x reference)
import functools

import jax
import jax.numpy as jnp
from jax import lax
from jax.experimental import pallas as pl
from jax.experimental.pallas import tpu as pltpu
from jax.experimental.pallas import tpu_sc as plsc

_BATCH = 4096
_EMB = 16
_NU = 2
_NI = 10
_LAYER = 256
_XDIM = 15448
_TOT = _NU * _EMB + _NI * _EMB + _XDIM

_NC = 2
_NS = 16
_NW = _NC * _NS

_CHUNK = 128

_UB = _BATCH * _NU
_IB = _BATCH * _NI
_U_PER = _UB // _NW
_I_PER = _IB // _NW
_UC = _U_PER // _CHUNK
_IC = _I_PER // _CHUNK


def _sc_gather_body(u_idx, i_idx, u_tab, i_tab, u_out, i_out,
                    uidx_v, urows_v, iidx_v, irows_v, sem):
    wid = lax.axis_index("s") * _NC + lax.axis_index("c")
    pltpu.sync_copy(u_idx.at[pl.ds(wid * _U_PER, _U_PER)], uidx_v)
    pltpu.sync_copy(i_idx.at[pl.ds(wid * _I_PER, _I_PER)], iidx_v)
    cps = []
    for j in range(_UC):
        cps.append(pltpu.async_copy(
            u_tab.at[uidx_v.at[pl.ds(j * _CHUNK, _CHUNK)]],
            urows_v.at[pl.ds(j * _CHUNK, _CHUNK)], sem))
    for j in range(_IC):
        cps.append(pltpu.async_copy(
            i_tab.at[iidx_v.at[pl.ds(j * _CHUNK, _CHUNK)]],
            irows_v.at[pl.ds(j * _CHUNK, _CHUNK)], sem))
    for cp in cps:
        cp.wait()
    pltpu.sync_copy(urows_v, u_out.at[pl.ds(wid * _U_PER, _U_PER)])
    pltpu.sync_copy(irows_v, i_out.at[pl.ds(wid * _I_PER, _I_PER)])


@functools.lru_cache(maxsize=1)
def _sc_gather():
    return pl.kernel(
        _sc_gather_body,
        mesh=plsc.VectorSubcoreMesh(core_axis_name="c", subcore_axis_name="s"),
        out_type=[
            jax.ShapeDtypeStruct((_UB, _EMB), jnp.float32),
            jax.ShapeDtypeStruct((_IB, _EMB), jnp.float32),
        ],
        scratch_types=[
            pltpu.VMEM((_U_PER,), jnp.int32),
            pltpu.VMEM((_U_PER, _EMB), jnp.float32),
            pltpu.VMEM((_I_PER,), jnp.int32),
            pltpu.VMEM((_I_PER, _EMB), jnp.float32),
            pltpu.SemaphoreType.DMA,
        ],
        compiler_params=pltpu.CompilerParams(use_tc_tiling_on_sc=False),
    )


_TK = 1024
_KT = 16
_KLAST = _XDIM - (_KT - 1) * _TK


def _p1_body(xt_ref, w1x_ref, h1_ref):
    k = pl.program_id(0)

    def _acc(p):
        @pl.when(k == 0)
        def _():
            h1_ref[...] = p

        @pl.when(k != 0)
        def _():
            h1_ref[...] += p

    @pl.when(k != _KT - 1)
    def _():
        _acc(lax.dot_general(
            xt_ref[...], w1x_ref[...],
            dimension_numbers=(((0,), (0,)), ((), ())),
            preferred_element_type=jnp.float32))

    @pl.when(k == _KT - 1)
    def _():
        valid = lax.broadcasted_iota(jnp.int32, (_TK, 1), 0) < _KLAST
        xm = jnp.where(valid, xt_ref[...], 0.0)
        wm = jnp.where(valid, w1x_ref[...], 0.0)
        _acc(lax.dot_general(
            xm, wm,
            dimension_numbers=(((0,), (0,)), ((), ())),
            preferred_element_type=jnp.float32))


def _p1(xt, W1x):
    return pl.pallas_call(
        _p1_body,
        grid=(_KT,),
        in_specs=[
            pl.BlockSpec((_TK, _BATCH), lambda k: (k, 0)),
            pl.BlockSpec((_TK, _LAYER), lambda k: (k, 0)),
        ],
        out_specs=pl.BlockSpec((_BATCH, _LAYER), lambda k: (0, 0)),
        out_shape=jax.ShapeDtypeStruct((_BATCH, _LAYER), jnp.float32),
        compiler_params=pltpu.CompilerParams(
            vmem_limit_bytes=100 * 1024 * 1024),
    )(xt, W1x)


def _p2_body(h1x_ref, ue_ref, ie_ref, w1e_ref, g1_ref, be1_ref,
             w2_ref, g2_ref, be2_ref, w3_ref, b3_ref, out_ref):
    nu = _NU * _EMB
    h1 = (h1x_ref[...]
          + jnp.dot(ue_ref[...], w1e_ref[:nu, :],
                    preferred_element_type=jnp.float32)
          + jnp.dot(ie_ref[...], w1e_ref[nu:, :],
                    preferred_element_type=jnp.float32))
    m1 = jnp.mean(h1, axis=0, keepdims=True)
    v1 = jnp.mean((h1 - m1) * (h1 - m1), axis=0, keepdims=True)
    h = (h1 - m1) * lax.rsqrt(v1 + 1e-5) * g1_ref[...] + be1_ref[...]
    h = jnp.maximum(h, 0.0)
    h2 = jnp.dot(h, w2_ref[...], preferred_element_type=jnp.float32)
    m2 = jnp.mean(h2, axis=0, keepdims=True)
    v2 = jnp.mean((h2 - m2) * (h2 - m2), axis=0, keepdims=True)
    h2 = (h2 - m2) * lax.rsqrt(v2 + 1e-5) * g2_ref[...] + be2_ref[...]
    h2 = jnp.maximum(h2, 0.0)
    out_ref[...] = (jnp.dot(h2, w3_ref[...],
                            preferred_element_type=jnp.float32)
                    + b3_ref[...])


def _p2(h1x, ue, ie, W1e, g1, be1, W2, g2, be2, W3, b3):
    full = lambda s: pl.BlockSpec(s, lambda: (0,) * len(s))
    return pl.pallas_call(
        _p2_body,
        in_specs=[
            full((_BATCH, _LAYER)),
            full((_BATCH, _NU * _EMB)),
            full((_BATCH, _NI * _EMB)),
            full((_NU * _EMB + _NI * _EMB, _LAYER)),
            full((1, _LAYER)),
            full((1, _LAYER)),
            full((_LAYER, _LAYER)),
            full((1, _LAYER)),
            full((1, _LAYER)),
            full((_LAYER, _NI)),
            full((1, _NI)),
        ],
        out_specs=full((_BATCH, _NI)),
        out_shape=jax.ShapeDtypeStruct((_BATCH, _NI), jnp.float32),
        compiler_params=pltpu.CompilerParams(
            vmem_limit_bytes=100 * 1024 * 1024),
    )(h1x, ue, ie, W1e, g1, be1, W2, g2, be2, W3, b3)


def kernel(x, u_cats, i_cats, user_table, item_table,
           W1, b1, g1, be1, W2, b2, g2, be2, W3, b3):
    u_idx = u_cats.reshape(_UB)
    i_idx = i_cats.reshape(_IB)
    u_rows, i_rows = _sc_gather()(u_idx, i_idx, user_table, item_table)
    ue = u_rows.reshape(_BATCH, _NU * _EMB)
    ie = i_rows.reshape(_BATCH, _NI * _EMB)
    nde = _NU * _EMB + _NI * _EMB
    h1x = _p1(x.T, W1[nde:, :])
    return _p2(h1x, ue, ie, W1[:nde, :],
               g1.reshape(1, _LAYER), be1.reshape(1, _LAYER),
               W2, g2.reshape(1, _LAYER), be2.reshape(1, _LAYER),
               W3, b3.reshape(1, _NI))
